# baseline (speedup 1.0000x reference)
import functools
import jax
import jax.numpy as jnp
import numpy as np
from jax import lax
from jax.experimental import pallas as pl
from jax.experimental.pallas import tpu as pltpu
from jax.experimental.pallas import tpu_sc as plsc

B = 1024
N_FIELDS = 26
EMB_DIM = 32
D_CAT = 224
TRIU = D_CAT * (D_CAT + 1) // 2
W1_COLS = TRIU + 128
CHUNK = 512
N_CHUNKS = 49
STREAM_END = N_CHUNKS * CHUNK

N_IDX = B * N_FIELDS
IDX_CHUNK = 104
N_WORKERS = 32
IDX_PER_W = N_IDX // N_WORKERS
CHUNKS_PER_W = IDX_PER_W // IDX_CHUNK
IDX_ROWS = N_IDX // IDX_CHUNK


def _row_off(i):
    return D_CAT * i - (i * (i - 1)) // 2


def _triu_indices():
    iu = np.empty((TRIU,), np.int32)
    ju = np.empty((TRIU,), np.int32)
    k = 0
    for i in range(D_CAT):
        w = D_CAT - i
        iu[k:k + w] = i
        ju[k:k + w] = np.arange(i, D_CAT, dtype=np.int32)
        k += w
    return iu, ju


_IU, _JU = _triu_indices()
_IU3 = _IU[:STREAM_END].reshape(N_CHUNKS, 1, CHUNK)
_JU3 = _JU[:STREAM_END].reshape(N_CHUNKS, 1, CHUNK)
_TAIL_SEGS = []
for _i in range(D_CAT):
    _lo = max(_row_off(_i), STREAM_END)
    _hi = min(_row_off(_i) + D_CAT - _i, TRIU)
    if _lo < _hi:
        _TAIL_SEGS.append((_i, _i + (_lo - _row_off(_i)), _i + (_hi - _row_off(_i))))


def _sc_gather(table, idx2d):
    mesh = plsc.VectorSubcoreMesh(core_axis_name="c", subcore_axis_name="s")

    @functools.partial(
        pl.kernel,
        mesh=mesh,
        compiler_params=pltpu.CompilerParams(use_tc_tiling_on_sc=False),
        out_type=jax.ShapeDtypeStruct((N_IDX, EMB_DIM), jnp.float32),
        scratch_types=[
            pltpu.VMEM((CHUNKS_PER_W, IDX_CHUNK), jnp.int32),
            pltpu.VMEM((IDX_PER_W, EMB_DIM), jnp.float32),
            pltpu.SemaphoreType.DMA,
        ],
    )
    def k(table_hbm, idx_hbm, out_hbm, idx_v, rows_v, sem):
        info = plsc.get_sparse_core_info()
        nc = info.num_cores
        wid = lax.axis_index("s") * nc + lax.axis_index("c")
        pltpu.sync_copy(idx_hbm.at[pl.ds(wid * CHUNKS_PER_W, CHUNKS_PER_W)],
                        idx_v)
        copies = []
        for j in range(CHUNKS_PER_W):
            copies.append(pltpu.make_async_copy(
                table_hbm.at[idx_v.at[j]],
                rows_v.at[pl.ds(j * IDX_CHUNK, IDX_CHUNK)],
                sem))
        for c in copies:
            c.start()
        for c in copies:
            c.wait()
        pltpu.sync_copy(rows_v, out_hbm.at[pl.ds(wid * IDX_PER_W, IDX_PER_W)])

    return k(table, idx2d)


def _dot_t(x, w):
    return lax.dot_general(x, w, (((1,), (1,)), ((), ())),
                           preferred_element_type=jnp.float32)


def _tc_body(rows_ref, xe_ref, xd_ref,
             pj_w1_ref, pj_b1_ref, pj_w2_ref, pj_b2_ref, pj_g_ref, pj_bt_ref,
             bm_w1_ref, bm_b1_ref, bm_w2_ref, bm_b2_ref,
             w1_ref, w1tail_ref, tm_b1_ref, tm_w2_ref, tm_b2_ref,
             tm_w3_ref, tm_b3_ref, iu_ref, ju_ref,
             out_ref, xemb_ref,
             c_s, bot_s, acc_ref):
    g = pl.program_id(0)

    @pl.when(g == 0)
    def _prologue():
        h = jnp.maximum(_dot_t(xe_ref[...], pj_w1_ref[...]) + pj_b1_ref[...],
                        0.0)
        h = _dot_t(h, pj_w2_ref[...]) + pj_b2_ref[...]
        mean = jnp.mean(h, axis=0, keepdims=True)
        var = jnp.mean((h - mean) * (h - mean), axis=0, keepdims=True)
        x_embed = (pj_g_ref[...] * (h - mean) * lax.rsqrt(var + 1e-5)
                   + pj_bt_ref[...])
        xemb_ref[...] = x_embed

        bot = jnp.maximum(_dot_t(xd_ref[...], bm_w1_ref[...])
                          + bm_b1_ref[...], 0.0)
        bot = jnp.maximum(_dot_t(bot, bm_w2_ref[...]) + bm_b2_ref[...], 0.0)
        bot_s[...] = bot

        r_mod = jax.lax.broadcasted_iota(jnp.int32,
                                         (N_FIELDS * EMB_DIM, EMB_DIM), 0)
        c_id = jax.lax.broadcasted_iota(jnp.int32,
                                        (N_FIELDS * EMB_DIM, EMB_DIM), 1)
        sel = (r_mod % EMB_DIM == c_id).astype(jnp.float32)
        embed_x = lax.dot_general(rows_ref[...], sel,
                                  (((1,), (0,)), ((), ())),
                                  preferred_element_type=jnp.float32)

        c = jnp.concatenate(
            [bot, x_embed, embed_x,
             jnp.zeros((B, 256 - D_CAT), jnp.float32)], axis=1)
        c_s[...] = c

        segs = [c[:, i:i + 1] * c[:, jl:jh] for i, jl, jh in _TAIL_SEGS]
        segs.append(bot)
        tail_prod = jnp.concatenate(segs, axis=1)
        acc_ref[...] = _dot_t(tail_prod, w1tail_ref[...]) + tm_b1_ref[...]

    c0 = c_s[:, 0:128]
    c1 = c_s[:, 128:256]
    iu = jnp.broadcast_to(iu_ref[0, 0, :][None, :], (B, CHUNK))
    ju = jnp.broadcast_to(ju_ref[0, 0, :][None, :], (B, CHUNK))
    iu_m = iu & 127
    ju_m = ju & 127
    cu = jnp.where(iu < 128,
                   jnp.take_along_axis(c0, iu_m, axis=1),
                   jnp.take_along_axis(c1, iu_m, axis=1))
    cv = jnp.where(ju < 128,
                   jnp.take_along_axis(c0, ju_m, axis=1),
                   jnp.take_along_axis(c1, ju_m, axis=1))
    acc_ref[...] = acc_ref[...] + _dot_t(cu * cv, w1_ref[...])

    @pl.when(g == N_CHUNKS - 1)
    def _epilogue():
        t = jnp.maximum(acc_ref[...], 0.0)
        t = jnp.maximum(_dot_t(t, tm_w2_ref[...]) + tm_b2_ref[...], 0.0)
        logit = _dot_t(t, tm_w3_ref[...])[:, 0:1] + tm_b3_ref[0, 0]
        out_ref[...] = jax.nn.sigmoid(logit)


def kernel(x_sparse, x_dense, x_embed_before_projection, emb_table,
           pj_w1, pj_b1, pj_w2, pj_b2, pj_gamma, pj_beta,
           bm_w1, bm_b1, bm_w2, bm_b2,
           tm_w1, tm_b1, tm_w2, tm_b2, tm_w3, tm_b3):
    idx2d = x_sparse.astype(jnp.int32).reshape(IDX_ROWS, IDX_CHUNK)
    rows = _sc_gather(emb_table, idx2d)
    rows832 = rows.reshape(B, N_FIELDS * EMB_DIM)

    def full(shape):
        nd = len(shape)
        return pl.BlockSpec(shape, lambda g, _nd=nd: (0,) * _nd)

    in_specs = [
        full((B, N_FIELDS * EMB_DIM)), full((B, 512)), full((B, 256)),
        full((256, 512)), full((1, 256)), full((64, 256)), full((1, 64)),
        full((1, 64)), full((1, 64)),
        full((256, 256)), full((1, 256)), full((128, 256)), full((1, 128)),
        pl.BlockSpec((512, CHUNK), lambda g: (0, g)),
        full((512, 240)), full((1, 512)), full((256, 512)), full((1, 256)),
        full((8, 256)),
        pl.BlockSpec(memory_space=pltpu.SMEM),
        pl.BlockSpec((1, 1, CHUNK), lambda g: (g, 0, 0)),
        pl.BlockSpec((1, 1, CHUNK), lambda g: (g, 0, 0)),
    ]

    out, xemb = pl.pallas_call(
        _tc_body,
        grid=(N_CHUNKS,),
        out_shape=(jax.ShapeDtypeStruct((B, 1), jnp.float32),
                   jax.ShapeDtypeStruct((B, 64), jnp.float32)),
        in_specs=in_specs,
        out_specs=(pl.BlockSpec((B, 1), lambda g: (0, 0)),
                   pl.BlockSpec((B, 64), lambda g: (0, 0))),
        scratch_shapes=[
            pltpu.VMEM((B, 256), jnp.float32),
            pltpu.VMEM((B, 128), jnp.float32),
            pltpu.VMEM((B, 512), jnp.float32),
        ],
    )(rows832, x_embed_before_projection, x_dense,
      pj_w1, pj_b1.reshape(1, -1), pj_w2, pj_b2.reshape(1, -1),
      pj_gamma.reshape(1, -1), pj_beta.reshape(1, -1),
      bm_w1, bm_b1.reshape(1, -1), bm_w2, bm_b2.reshape(1, -1),
      tm_w1, tm_w1[:, STREAM_END:], tm_b1.reshape(1, -1),
      tm_w2, tm_b2.reshape(1, -1), jnp.pad(tm_w3, ((0, 7), (0, 0))),
      tm_b3.reshape(1, 1), jnp.asarray(_IU3), jnp.asarray(_JU3))
    return (out, xemb)
